# baseline (speedup 1.0000x reference)
import functools

import jax
import jax.numpy as jnp
from jax import lax
from jax.experimental import pallas as pl
from jax.experimental.pallas import tpu as pltpu
from jax.experimental.pallas import tpu_sc as plsc

N = 10000
E = 320000
D_IN = 128
D_HID = 16
D_OUT = 64

NC = 2
NS = 16
NW = NC * NS

CHUNK = 128
CPT = 80
EPT = CPT * CHUNK
E_PAD = EPT * NW
E_ROWS = E_PAD // CHUNK

NACC = 10112
STRIPE = NACC // NS
DUMMY = N

def _deg_body(src_hbm, dst_hbm, zeros_hbm, ones_hbm,
              deg_hbm, dstp_hbm,
              idx_s, idx_d, idx_p, ones_v, acc, sem):
    c = lax.axis_index("c")
    s = lax.axis_index("s")
    tid = c * NS + s
    base_row = tid * CPT
    pltpu.sync_copy(zeros_hbm, acc.at[pl.ds(s * STRIPE, STRIPE)])
    pltpu.sync_copy(ones_hbm, ones_v)
    pltpu.sync_copy(src_hbm.at[pl.ds(base_row, CPT)], idx_s)
    pltpu.sync_copy(dst_hbm.at[pl.ds(base_row, CPT)], idx_d)
    plsc.subcore_barrier()

    def chunk(j, carry):
        for k in range(CHUNK // 16):
            sv = idx_s[j, pl.ds(k * 16, 16)]
            dv = idx_d[j, pl.ds(k * 16, 16)]
            idx_p[j, pl.ds(k * 16, 16)] = jnp.where(sv == dv, jnp.int32(DUMMY), dv)
        pltpu.sync_copy(ones_v, acc.at[idx_p.at[j]], add=True)
        return carry

    lax.fori_loop(0, CPT, chunk, 0)
    pltpu.sync_copy(idx_p, dstp_hbm.at[pl.ds(base_row, CPT)])
    plsc.subcore_barrier()
    pltpu.sync_copy(acc.at[pl.ds(s * STRIPE, STRIPE)],
                    deg_hbm.at[c, pl.ds(s * STRIPE, STRIPE)])


NB = 4
TSTEP = CPT // (2 * NB)


def _prop_body(g_hbm, src_hbm, dstp_hbm, zeros_hbm,
               out_hbm,
               idx_s, idx_d, rows, acc, gsA, gsB, ssA, ssB):
    c = lax.axis_index("c")
    s = lax.axis_index("s")
    tid = c * NS + s
    base_row = tid * CPT
    pltpu.sync_copy(zeros_hbm, acc.at[pl.ds(s * STRIPE, STRIPE)])
    pltpu.sync_copy(src_hbm.at[pl.ds(base_row, CPT)], idx_s.at[pl.ds(0, CPT)])
    pltpu.sync_copy(dstp_hbm.at[pl.ds(base_row, CPT)], idx_d)
    zero16 = jnp.zeros((16,), jnp.int32)
    for r in range(NB):
        for k in range(CHUNK // 16):
            idx_s[CPT + r, pl.ds(k * 16, 16)] = zero16
    plsc.subcore_barrier()

    def gather(j, b, sem):
        return pltpu.async_copy(g_hbm.at[idx_s.at[j]], rows.at[b], sem)

    def gather_wait(j, b, sem):
        pltpu.make_async_copy(g_hbm.at[idx_s.at[j]], rows.at[b], sem).wait()

    def scatter(j, b, sem):
        return pltpu.async_copy(rows.at[b], acc.at[idx_d.at[j]], sem, add=True)

    for b in range(NB):
        gather(b, b, gsA)

    def body(tt, carry):
        jA = 2 * NB * tt
        jB = jA + NB
        jN = jA + 2 * NB
        for b in range(NB):
            gather_wait(jA + b, b, gsA)
        sA = [scatter(jA + b, b, ssA) for b in range(NB)]
        gB = [gather(jB + b, NB + b, gsB) for b in range(NB)]
        for d in sA:
            d.wait()
        for d in gB:
            d.wait()
        sB = [scatter(jB + b, NB + b, ssB) for b in range(NB)]
        for b in range(NB):
            gather(jN + b, b, gsA)
        for d in sB:
            d.wait()
        return carry

    lax.fori_loop(0, TSTEP, body, 0)
    for b in range(NB):
        gather_wait(CPT + b, b, gsA)
    plsc.subcore_barrier()
    pltpu.sync_copy(acc.at[pl.ds(s * STRIPE, STRIPE)],
                    out_hbm.at[c, pl.ds(s * STRIPE, STRIPE)])


@functools.cache
def _sc_kernels():
    mesh = plsc.VectorSubcoreMesh(core_axis_name="c", subcore_axis_name="s",
                                  num_cores=NC, num_subcores=NS)
    params = pltpu.CompilerParams(use_tc_tiling_on_sc=False)
    deg_kernel = pl.kernel(
        _deg_body,
        out_type=(jax.ShapeDtypeStruct((NC, NACC, D_HID), jnp.float32),
                  jax.ShapeDtypeStruct((E_ROWS, CHUNK), jnp.int32)),
        mesh=mesh,
        scratch_types=[
            pltpu.VMEM((CPT, CHUNK), jnp.int32),
            pltpu.VMEM((CPT, CHUNK), jnp.int32),
            pltpu.VMEM((CPT, CHUNK), jnp.int32),
            pltpu.VMEM((CHUNK, D_HID), jnp.float32),
            pltpu.VMEM_SHARED((NACC, D_HID), jnp.float32),
            pltpu.SemaphoreType.DMA,
        ],
        compiler_params=params,
    )
    prop_kernel = pl.kernel(
        _prop_body,
        out_type=jax.ShapeDtypeStruct((NC, NACC, D_HID), jnp.float32),
        mesh=mesh,
        scratch_types=[
            pltpu.VMEM((CPT + NB, CHUNK), jnp.int32),
            pltpu.VMEM((CPT, CHUNK), jnp.int32),
            pltpu.VMEM((2 * NB, CHUNK, D_HID), jnp.float32),
            pltpu.VMEM_SHARED((NACC, D_HID), jnp.float32),
            pltpu.SemaphoreType.DMA,
            pltpu.SemaphoreType.DMA,
            pltpu.SemaphoreType.DMA,
            pltpu.SemaphoreType.DMA,
        ],
        compiler_params=params,
    )
    return deg_kernel, prop_kernel


_RB = 1000


def _lin1_body(deg2_ref, x_ref, w1_ref, dinv_ref, g1_ref):
    deg = deg2_ref[0] + deg2_ref[1] + 1.0
    dinv = lax.rsqrt(deg)
    dinv_ref[...] = dinv
    g1_ref[...] = dinv * jnp.dot(x_ref[...], w1_ref[...],
                                 preferred_element_type=jnp.float32)


def _lin1(deg2, x, W1):
    return pl.pallas_call(
        _lin1_body,
        grid=(N // _RB,),
        in_specs=[
            pl.BlockSpec((NC, _RB, D_HID), lambda i: (0, i, 0)),
            pl.BlockSpec((_RB, D_IN), lambda i: (i, 0)),
            pl.BlockSpec((D_IN, D_HID), lambda i: (0, 0)),
        ],
        out_specs=[
            pl.BlockSpec((_RB, D_HID), lambda i: (i, 0)),
            pl.BlockSpec((_RB, D_HID), lambda i: (i, 0)),
        ],
        out_shape=[jax.ShapeDtypeStruct((N, D_HID), jnp.float32),
                   jax.ShapeDtypeStruct((N, D_HID), jnp.float32)],
    )(deg2, x, W1)


def _hid_body(s1_ref, g1_ref, dinv_ref, b1_ref, g2_ref):
    pre = dinv_ref[...] * (s1_ref[0] + s1_ref[1] + g1_ref[...]) + b1_ref[...]
    g2_ref[...] = dinv_ref[...] * jnp.maximum(pre, 0.0)


def _hid(s1, g1, dinv, b1):
    return pl.pallas_call(
        _hid_body,
        grid=(N // _RB,),
        in_specs=[
            pl.BlockSpec((NC, _RB, D_HID), lambda i: (0, i, 0)),
            pl.BlockSpec((_RB, D_HID), lambda i: (i, 0)),
            pl.BlockSpec((_RB, D_HID), lambda i: (i, 0)),
            pl.BlockSpec((1, D_HID), lambda i: (0, 0)),
        ],
        out_specs=pl.BlockSpec((_RB, D_HID), lambda i: (i, 0)),
        out_shape=jax.ShapeDtypeStruct((N, D_HID), jnp.float32),
    )(s1, g1, dinv, b1)


def _out_body(s2_ref, g2_ref, dinv_ref, w2_ref, b2_ref, out_ref):
    h = dinv_ref[...] * (s2_ref[0] + s2_ref[1] + g2_ref[...])
    z = jnp.dot(h, w2_ref[...], preferred_element_type=jnp.float32) + b2_ref[...]
    m = jnp.max(z, axis=1, keepdims=True)
    lse = jnp.log(jnp.sum(jnp.exp(z - m), axis=1, keepdims=True)) + m
    out_ref[...] = z - lse


def _out(s2, g2, dinv, W2, b2):
    return pl.pallas_call(
        _out_body,
        grid=(N // _RB,),
        in_specs=[
            pl.BlockSpec((NC, _RB, D_HID), lambda i: (0, i, 0)),
            pl.BlockSpec((_RB, D_HID), lambda i: (i, 0)),
            pl.BlockSpec((_RB, D_HID), lambda i: (i, 0)),
            pl.BlockSpec((D_HID, D_OUT), lambda i: (0, 0)),
            pl.BlockSpec((1, D_OUT), lambda i: (0, 0)),
        ],
        out_specs=pl.BlockSpec((_RB, D_OUT), lambda i: (i, 0)),
        out_shape=jax.ShapeDtypeStruct((N, D_OUT), jnp.float32),
    )(s2, g2, dinv, W2, b2)


def kernel(x, edge_index, W1, b1, W2, b2):
    src = edge_index[0].astype(jnp.int32)
    dst = edge_index[1].astype(jnp.int32)
    pad = E_PAD - E
    src_p = jnp.concatenate([src, jnp.zeros((pad,), jnp.int32)]).reshape(E_ROWS, CHUNK)
    dst_p = jnp.concatenate([dst, jnp.full((pad,), DUMMY, jnp.int32)]).reshape(E_ROWS, CHUNK)

    zeros_stripe = jnp.zeros((STRIPE, D_HID), jnp.float32)
    ones_chunk = jnp.ones((CHUNK, D_HID), jnp.float32)

    deg_kernel, prop_kernel = _sc_kernels()
    deg2, dstp = deg_kernel(src_p, dst_p, zeros_stripe, ones_chunk)
    dinv, g1 = _lin1(deg2[:, :N], x, W1)

    s1 = prop_kernel(g1, src_p, dstp, zeros_stripe)
    g2 = _hid(s1[:, :N], g1, dinv, b1.reshape(1, D_HID))

    s2 = prop_kernel(g2, src_p, dstp, zeros_stripe)
    return _out(s2[:, :N], g2, dinv, W2, b2.reshape(1, D_OUT))


# speedup vs baseline: 1.3744x; 1.3744x over previous
import functools

import jax
import jax.numpy as jnp
from jax import lax
from jax.experimental import pallas as pl
from jax.experimental.pallas import tpu as pltpu
from jax.experimental.pallas import tpu_sc as plsc

N = 10000
E = 320000
D_IN = 128
D_HID = 16
D_OUT = 64

NC = 2
NS = 16
NW = NC * NS

CHUNK = 128
CPT = 80
EPT = CPT * CHUNK
E_PAD = EPT * NW
E_ROWS = E_PAD // CHUNK

NACC = 10112
STRIPE = NACC // NS
DUMMY = N

def _deg_body(src_hbm, dst_hbm, zeros_hbm, ones_hbm,
              deg_hbm, dstp_hbm,
              idx_s, idx_d, idx_p, ones_v, acc, sem):
    c = lax.axis_index("c")
    s = lax.axis_index("s")
    tid = c * NS + s
    base_row = tid * CPT
    pltpu.sync_copy(zeros_hbm, acc.at[pl.ds(s * STRIPE, STRIPE)])
    pltpu.sync_copy(ones_hbm, ones_v)
    pltpu.sync_copy(src_hbm.at[pl.ds(base_row, CPT)], idx_s)
    pltpu.sync_copy(dst_hbm.at[pl.ds(base_row, CPT)], idx_d)
    plsc.subcore_barrier()

    def chunk(j, carry):
        for k in range(CHUNK // 16):
            sv = idx_s[j, pl.ds(k * 16, 16)]
            dv = idx_d[j, pl.ds(k * 16, 16)]
            idx_p[j, pl.ds(k * 16, 16)] = jnp.where(sv == dv, jnp.int32(DUMMY), dv)
        pltpu.sync_copy(ones_v, acc.at[idx_p.at[j]], add=True)
        return carry

    lax.fori_loop(0, CPT, chunk, 0)
    pltpu.sync_copy(idx_p, dstp_hbm.at[pl.ds(base_row, CPT)])
    plsc.subcore_barrier()
    pltpu.sync_copy(acc.at[pl.ds(s * STRIPE, STRIPE)],
                    deg_hbm.at[c, pl.ds(s * STRIPE, STRIPE)])


NB = 2


def _prop_body(g_hbm, src_hbm, dstp_hbm, zeros_hbm,
               out_hbm,
               idx_s, idx_d, rows, acc, gs0, gs1):
    c = lax.axis_index("c")
    s = lax.axis_index("s")
    tid = c * NS + s
    base_row = tid * CPT
    pltpu.sync_copy(zeros_hbm, acc.at[pl.ds(s * STRIPE, STRIPE)])
    pltpu.sync_copy(src_hbm.at[pl.ds(base_row, CPT)], idx_s.at[pl.ds(0, CPT)])
    pltpu.sync_copy(dstp_hbm.at[pl.ds(base_row, CPT)], idx_d)
    zero16 = jnp.zeros((16,), jnp.int32)
    for k in range(CHUNK // 16):
        idx_s[CPT, pl.ds(k * 16, 16)] = zero16
    plsc.subcore_barrier()

    def gather(j, b, sem):
        return pltpu.async_copy(g_hbm.at[idx_s.at[j]], rows.at[b], sem)

    def gather_wait(j, b, sem):
        pltpu.make_async_copy(g_hbm.at[idx_s.at[j]], rows.at[b], sem).wait()

    gather(0, 0, gs0)

    def body(t, carry):
        j = 2 * t
        gather(j + 1, 1, gs1)
        gather_wait(j, 0, gs0)
        pltpu.sync_copy(rows.at[0], acc.at[idx_d.at[j]], add=True)
        gather(j + 2, 0, gs0)
        gather_wait(j + 1, 1, gs1)
        pltpu.sync_copy(rows.at[1], acc.at[idx_d.at[j + 1]], add=True)
        return carry

    lax.fori_loop(0, CPT // 2, body, 0)
    gather_wait(CPT, 0, gs0)
    plsc.subcore_barrier()
    pltpu.sync_copy(acc.at[pl.ds(s * STRIPE, STRIPE)],
                    out_hbm.at[c, pl.ds(s * STRIPE, STRIPE)])


@functools.cache
def _sc_kernels():
    mesh = plsc.VectorSubcoreMesh(core_axis_name="c", subcore_axis_name="s",
                                  num_cores=NC, num_subcores=NS)
    params = pltpu.CompilerParams(use_tc_tiling_on_sc=False)
    deg_kernel = pl.kernel(
        _deg_body,
        out_type=(jax.ShapeDtypeStruct((NC, NACC, D_HID), jnp.float32),
                  jax.ShapeDtypeStruct((E_ROWS, CHUNK), jnp.int32)),
        mesh=mesh,
        scratch_types=[
            pltpu.VMEM((CPT, CHUNK), jnp.int32),
            pltpu.VMEM((CPT, CHUNK), jnp.int32),
            pltpu.VMEM((CPT, CHUNK), jnp.int32),
            pltpu.VMEM((CHUNK, D_HID), jnp.float32),
            pltpu.VMEM_SHARED((NACC, D_HID), jnp.float32),
            pltpu.SemaphoreType.DMA,
        ],
        compiler_params=params,
    )
    prop_kernel = pl.kernel(
        _prop_body,
        out_type=jax.ShapeDtypeStruct((NC, NACC, D_HID), jnp.float32),
        mesh=mesh,
        scratch_types=[
            pltpu.VMEM((CPT + 8, CHUNK), jnp.int32),
            pltpu.VMEM((CPT, CHUNK), jnp.int32),
            pltpu.VMEM((NB, CHUNK, D_HID), jnp.float32),
            pltpu.VMEM_SHARED((NACC, D_HID), jnp.float32),
            pltpu.SemaphoreType.DMA,
            pltpu.SemaphoreType.DMA,
        ],
        compiler_params=params,
    )
    return deg_kernel, prop_kernel


_RB = 1000


def _lin1_body(deg2_ref, x_ref, w1_ref, dinv_ref, g1_ref):
    deg = deg2_ref[0] + deg2_ref[1] + 1.0
    dinv = lax.rsqrt(deg)
    dinv_ref[...] = dinv
    g1_ref[...] = dinv * jnp.dot(x_ref[...], w1_ref[...],
                                 preferred_element_type=jnp.float32)


def _lin1(deg2, x, W1):
    return pl.pallas_call(
        _lin1_body,
        grid=(N // _RB,),
        in_specs=[
            pl.BlockSpec((NC, _RB, D_HID), lambda i: (0, i, 0)),
            pl.BlockSpec((_RB, D_IN), lambda i: (i, 0)),
            pl.BlockSpec((D_IN, D_HID), lambda i: (0, 0)),
        ],
        out_specs=[
            pl.BlockSpec((_RB, D_HID), lambda i: (i, 0)),
            pl.BlockSpec((_RB, D_HID), lambda i: (i, 0)),
        ],
        out_shape=[jax.ShapeDtypeStruct((N, D_HID), jnp.float32),
                   jax.ShapeDtypeStruct((N, D_HID), jnp.float32)],
    )(deg2, x, W1)


def _hid_body(s1_ref, g1_ref, dinv_ref, b1_ref, g2_ref):
    pre = dinv_ref[...] * (s1_ref[0] + s1_ref[1] + g1_ref[...]) + b1_ref[...]
    g2_ref[...] = dinv_ref[...] * jnp.maximum(pre, 0.0)


def _hid(s1, g1, dinv, b1):
    return pl.pallas_call(
        _hid_body,
        grid=(N // _RB,),
        in_specs=[
            pl.BlockSpec((NC, _RB, D_HID), lambda i: (0, i, 0)),
            pl.BlockSpec((_RB, D_HID), lambda i: (i, 0)),
            pl.BlockSpec((_RB, D_HID), lambda i: (i, 0)),
            pl.BlockSpec((1, D_HID), lambda i: (0, 0)),
        ],
        out_specs=pl.BlockSpec((_RB, D_HID), lambda i: (i, 0)),
        out_shape=jax.ShapeDtypeStruct((N, D_HID), jnp.float32),
    )(s1, g1, dinv, b1)


def _out_body(s2_ref, g2_ref, dinv_ref, w2_ref, b2_ref, out_ref):
    h = dinv_ref[...] * (s2_ref[0] + s2_ref[1] + g2_ref[...])
    z = jnp.dot(h, w2_ref[...], preferred_element_type=jnp.float32) + b2_ref[...]
    m = jnp.max(z, axis=1, keepdims=True)
    lse = jnp.log(jnp.sum(jnp.exp(z - m), axis=1, keepdims=True)) + m
    out_ref[...] = z - lse


def _out(s2, g2, dinv, W2, b2):
    return pl.pallas_call(
        _out_body,
        grid=(N // _RB,),
        in_specs=[
            pl.BlockSpec((NC, _RB, D_HID), lambda i: (0, i, 0)),
            pl.BlockSpec((_RB, D_HID), lambda i: (i, 0)),
            pl.BlockSpec((_RB, D_HID), lambda i: (i, 0)),
            pl.BlockSpec((D_HID, D_OUT), lambda i: (0, 0)),
            pl.BlockSpec((1, D_OUT), lambda i: (0, 0)),
        ],
        out_specs=pl.BlockSpec((_RB, D_OUT), lambda i: (i, 0)),
        out_shape=jax.ShapeDtypeStruct((N, D_OUT), jnp.float32),
    )(s2, g2, dinv, W2, b2)


def kernel(x, edge_index, W1, b1, W2, b2):
    src = edge_index[0].astype(jnp.int32)
    dst = edge_index[1].astype(jnp.int32)
    pad = E_PAD - E
    src_p = jnp.concatenate([src, jnp.zeros((pad,), jnp.int32)]).reshape(E_ROWS, CHUNK)
    dst_p = jnp.concatenate([dst, jnp.full((pad,), DUMMY, jnp.int32)]).reshape(E_ROWS, CHUNK)

    zeros_stripe = jnp.zeros((STRIPE, D_HID), jnp.float32)
    ones_chunk = jnp.ones((CHUNK, D_HID), jnp.float32)

    deg_kernel, prop_kernel = _sc_kernels()
    deg2, dstp = deg_kernel(src_p, dst_p, zeros_stripe, ones_chunk)
    dinv, g1 = _lin1(deg2[:, :N], x, W1)

    s1 = prop_kernel(g1, src_p, dstp, zeros_stripe)
    g2 = _hid(s1[:, :N], g1, dinv, b1.reshape(1, D_HID))

    s2 = prop_kernel(g2, src_p, dstp, zeros_stripe)
    return _out(s2[:, :N], g2, dinv, W2, b2.reshape(1, D_OUT))
